# baseline (speedup 1.0000x reference)
import jax
import jax.numpy as jnp
from jax import lax
from jax.experimental import pallas as pl
from jax.experimental.pallas import tpu as pltpu
from jax.experimental.pallas import tpu_sc as plsc

_NUM_CORES = 2
_NUM_SUBCORES = 16
_NW = _NUM_CORES * _NUM_SUBCORES

_B = 1024 * 50
_D = 1024
_BPW = _B // _NW
_C = 40
_NCHUNK = _BPW // _C


def _body(idx_hbm, table_hbm, out_hbm, idx_v, buf_v, gsem):
    wid = lax.axis_index("s") * _NUM_CORES + lax.axis_index("c")
    base = wid * _BPW
    pltpu.sync_copy(idx_hbm.at[pl.ds(base, _BPW)], idx_v)
    for i in range(_NCHUNK):
        pltpu.async_copy(
            table_hbm.at[idx_v.at[pl.ds(i * _C, _C)]], buf_v, gsem
        ).wait()
        pltpu.sync_copy(buf_v, out_hbm.at[pl.ds(base + i * _C, _C)])


@jax.jit
def _lookup(indices_flat, table):
    mesh = plsc.VectorSubcoreMesh(core_axis_name="c", subcore_axis_name="s")
    f = pl.kernel(
        _body,
        out_type=jax.ShapeDtypeStruct((_B, _D), jnp.float32),
        mesh=mesh,
        scratch_types=[
            pltpu.VMEM((_BPW,), jnp.int32),
            pltpu.VMEM((_C, _D), jnp.float32),
            pltpu.SemaphoreType.DMA,
        ],
    )
    return f(indices_flat, table)


def kernel(indices, embedding_weight):
    b, t = indices.shape
    flat = indices.reshape(-1).astype(jnp.int32)
    out = _lookup(flat, embedding_weight)
    return out.reshape(b, t, _D)
